# baseline (speedup 1.0000x reference)
import functools

import jax
import jax.numpy as jnp
from jax import lax
from jax.experimental import pallas as pl
from jax.experimental.pallas import tpu as pltpu
from jax.experimental.pallas import tpu_sc as plsc

NDIM = 3
KS = 3
K = KS ** NDIM
KP = 32
DILATION = 0.05
MAX_DISTANCE = 1.0
SENT_VAL = 1e30

NTILES = 32
LANES = 16

CHUNK = 64
VPC = CHUNK // LANES
IDX_PER_CHUNK = CHUNK // LANES * 8 * K * LANES
GATHER_W = 128
NSTREAM = IDX_PER_CHUNK // GATHER_W


def _sc_kernel_body(nchunks, px_hbm, py_hbm, pz_hbm, atab_hbm, btab_hbm,
                    stab_hbm, dtab_hbm, sdf_hbm, out_hbm,
                    pxv, pyv, pzv, atab_v, btab_v, stab_v, dtab_v,
                    idx_buf, val_buf, cur_buf, gsem):
    M = 8
    wid = lax.axis_index("s") * 2 + lax.axis_index("c")
    b = wid // 8
    pbase = wid * (nchunks * CHUNK)

    pltpu.sync_copy(atab_hbm.at[pl.ds(b * (M * 16 * LANES), M * 16 * LANES)], atab_v)
    pltpu.sync_copy(btab_hbm.at[pl.ds(b * (M * 4 * LANES), M * 4 * LANES)], btab_v)
    pltpu.sync_copy(stab_hbm.at[pl.ds(b * (M * LANES), M * LANES)], stab_v)
    pltpu.sync_copy(dtab_hbm.at[pl.ds(b * (M * K * 3 * LANES), M * K * 3 * LANES)], dtab_v)

    zeros = jnp.zeros((LANES,), jnp.float32)

    def _zero(i, _):
        cur_buf[pl.ds(i * LANES, LANES)] = zeros
        return 0

    lax.fori_loop(0, CHUNK * KP // LANES, _zero, 0)

    iota = lax.iota(jnp.int32, LANES)
    iota_kp = iota * KP

    def chunk_body(g, _):
        off = pbase + g * CHUNK
        pltpu.sync_copy(px_hbm.at[pl.ds(off, CHUNK)], pxv)
        pltpu.sync_copy(py_hbm.at[pl.ds(off, CHUNK)], pyv)
        pltpu.sync_copy(pz_hbm.at[pl.ds(off, CHUNK)], pzv)

        def va_body(v, _):
            px = pxv[pl.ds(v * LANES, LANES)]
            py = pyv[pl.ds(v * LANES, LANES)]
            pz = pzv[pl.ds(v * LANES, LANES)]

            def m_body(m, _):
                ao = m * 16 * LANES
                a00 = atab_v[pl.ds(ao + 0 * LANES, LANES)]
                a01 = atab_v[pl.ds(ao + 1 * LANES, LANES)]
                a02 = atab_v[pl.ds(ao + 2 * LANES, LANES)]
                a10 = atab_v[pl.ds(ao + 3 * LANES, LANES)]
                a11 = atab_v[pl.ds(ao + 4 * LANES, LANES)]
                a12 = atab_v[pl.ds(ao + 5 * LANES, LANES)]
                a20 = atab_v[pl.ds(ao + 6 * LANES, LANES)]
                a21 = atab_v[pl.ds(ao + 7 * LANES, LANES)]
                a22 = atab_v[pl.ds(ao + 8 * LANES, LANES)]
                c0 = atab_v[pl.ds(ao + 9 * LANES, LANES)]
                c1 = atab_v[pl.ds(ao + 10 * LANES, LANES)]
                c2 = atab_v[pl.ds(ao + 11 * LANES, LANES)]
                dxf = atab_v[pl.ds(ao + 12 * LANES, LANES)]
                dyf = atab_v[pl.ds(ao + 13 * LANES, LANES)]
                dzf = atab_v[pl.ds(ao + 14 * LANES, LANES)]
                bo = m * 4 * LANES
                dyi = btab_v[pl.ds(bo + 0 * LANES, LANES)]
                dzi = btab_v[pl.ds(bo + 1 * LANES, LANES)]
                basev = btab_v[pl.ds(bo + 2 * LANES, LANES)]
                sentv = btab_v[pl.ds(bo + 3 * LANES, LANES)]

                xc = px * a00 + py * a01 + pz * a02 + c0
                yc = px * a10 + py * a11 + pz * a12 + c1
                zc = px * a20 + py * a21 + pz * a22 + c2

                do = m * (K * 3 * LANES)
                qbase = v * (8 * K * LANES) + m * (K * LANES)
                for k in range(K):
                    dx = dtab_v[pl.ds(do + (k * 3 + 0) * LANES, LANES)]
                    dy = dtab_v[pl.ds(do + (k * 3 + 1) * LANES, LANES)]
                    dz = dtab_v[pl.ds(do + (k * 3 + 2) * LANES, LANES)]
                    ux = xc + dx
                    uy = yc + dy
                    uz = zc + dz
                    inb = ((ux >= 0.0) & (ux < dxf)) & ((uy >= 0.0) & (uy < dyf))
                    inb = inb & ((uz >= 0.0) & (uz < dzf))
                    gx = ux.astype(jnp.int32)
                    gy = uy.astype(jnp.int32)
                    gz = uz.astype(jnp.int32)
                    flat = (gx * dyi + gy) * dzi + gz + basev
                    idx = jnp.where(inb, flat, sentv)
                    idx_buf[pl.ds(qbase + k * LANES, LANES)] = idx
                return 0

            lax.fori_loop(0, M, m_body, 0)
            return 0

        lax.fori_loop(0, VPC, va_body, 0)

        def fire(j, _):
            pltpu.async_copy(
                sdf_hbm.at[idx_buf.at[pl.ds(j * GATHER_W, GATHER_W)]],
                val_buf.at[pl.ds(j * GATHER_W, GATHER_W)], gsem)
            return 0

        lax.fori_loop(0, NSTREAM, fire, 0)

        def drain(j, _):
            pltpu.make_async_copy(
                sdf_hbm.at[idx_buf.at[pl.ds(j * GATHER_W, GATHER_W)]],
                val_buf.at[pl.ds(j * GATHER_W, GATHER_W)], gsem).wait()
            return 0

        lax.fori_loop(0, NSTREAM, drain, 0)

        def vb_body(v, _):
            svecs = [stab_v[pl.ds(m * LANES, LANES)] for m in range(M)]
            col = iota_kp + v * (LANES * KP)
            maxd = jnp.full((LANES,), MAX_DISTANCE, jnp.float32)
            for k in range(K):
                cur = maxd
                for m in range(M):
                    vo = v * (M * K * LANES) + m * (K * LANES) + k * LANES
                    cur = jnp.minimum(cur, val_buf[pl.ds(vo, LANES)] * svecs[m])
                plsc.store_scatter(cur_buf, [col + k], cur)
            return 0

        lax.fori_loop(0, VPC, vb_body, 0)

        pltpu.sync_copy(cur_buf, out_hbm.at[pl.ds((pbase + g * CHUNK) * KP,
                                                  CHUNK * KP)])
        return 0

    lax.fori_loop(0, nchunks, chunk_body, 0)


def _tc_matmul_body(x_ref, w_ref, b_ref, o_ref):
    o_ref[...] = jnp.dot(x_ref[...], w_ref[...],
                         preferred_element_type=jnp.float32) + b_ref[...]


def kernel(locs, idxs, poses, scales, sdf_data, sdf_offsets, sdf_shapes, weight, bias):
    B, N, _ = locs.shape
    M = idxs.shape[1]
    O = bias.shape[0]

    tiles_per_b = NTILES // B
    npad = ((N + tiles_per_b * CHUNK - 1) // (tiles_per_b * CHUNK)) * (tiles_per_b * CHUNK)
    nchunks = npad // (tiles_per_b * CHUNK)
    tp = B * npad

    p3 = jnp.pad(locs[..., :NDIM], ((0, 0), (0, npad - N), (0, 0)))
    px = p3[..., 0].reshape(-1)
    py = p3[..., 1].reshape(-1)
    pz = p3[..., 2].reshape(-1)

    t = poses[..., :NDIM]
    q = poses[..., NDIM:NDIM + 4]
    u = -q[..., :3]
    qw = q[..., 3]
    ux_, uy_, uz_ = u[..., 0], u[..., 1], u[..., 2]
    n2 = ux_ * ux_ + uy_ * uy_ + uz_ * uz_
    r00 = 1.0 - 2.0 * n2 + 2.0 * ux_ * ux_
    r11 = 1.0 - 2.0 * n2 + 2.0 * uy_ * uy_
    r22 = 1.0 - 2.0 * n2 + 2.0 * uz_ * uz_
    r01 = 2.0 * ux_ * uy_ - 2.0 * qw * uz_
    r02 = 2.0 * ux_ * uz_ + 2.0 * qw * uy_
    r10 = 2.0 * uy_ * ux_ + 2.0 * qw * uz_
    r12 = 2.0 * uy_ * uz_ - 2.0 * qw * ux_
    r20 = 2.0 * uz_ * ux_ - 2.0 * qw * uy_
    r21 = 2.0 * uz_ * uy_ + 2.0 * qw * ux_
    R = jnp.stack([r00, r01, r02, r10, r11, r12, r20, r21, r22],
                  axis=-1).reshape(B, M, 3, 3)

    sidx = idxs
    cell = sdf_shapes[sidx, NDIM]
    dims = sdf_shapes[sidx, :NDIM]
    base = sdf_offsets[sidx]
    inv = 1.0 / (scales * cell)

    A = R * inv[..., None, None]
    c = -jnp.sum(A * t[..., None, :], axis=-1)

    half = (KS - 1) // 2
    r = jnp.arange(-half, half + 1, dtype=jnp.float32) * DILATION
    ox, oy, oz = jnp.meshgrid(r, r, r, indexing='ij')
    offs = jnp.stack([ox.ravel(), oy.ravel(), oz.ravel()], axis=-1)
    d = jnp.sum(R[:, :, None, :, :] * offs[None, None, :, None, :], axis=-1)
    d = d * inv[..., None, None]

    atab = jnp.concatenate([A.reshape(B, M, 9), c, dims], axis=-1)
    atab = jnp.pad(atab, ((0, 0), (0, 0), (0, 1)))
    atab = jnp.broadcast_to(atab[..., None], (B, M, 16, LANES)).reshape(-1)

    sent = jnp.int32(sdf_data.shape[0])
    btab = jnp.stack([dims[..., 1].astype(jnp.int32),
                      dims[..., 2].astype(jnp.int32),
                      base.astype(jnp.int32),
                      jnp.broadcast_to(sent, (B, M))], axis=-1)
    btab = jnp.broadcast_to(btab[..., None], (B, M, 4, LANES)).reshape(-1)

    stab = jnp.broadcast_to(scales[..., None], (B, M, LANES)).reshape(-1)
    dtab = jnp.broadcast_to(d.reshape(B, M, K * 3)[..., None],
                            (B, M, K * 3, LANES)).reshape(-1)

    sdf_ext = jnp.concatenate(
        [sdf_data, jnp.full((16,), SENT_VAL, jnp.float32)])

    mesh = plsc.VectorSubcoreMesh(core_axis_name="c", subcore_axis_name="s")
    sc = pl.kernel(
        functools.partial(_sc_kernel_body, nchunks),
        out_type=jax.ShapeDtypeStruct((tp * KP,), jnp.float32),
        mesh=mesh,
        compiler_params=pltpu.CompilerParams(needs_layout_passes=False),
        scratch_types=[
            pltpu.VMEM((CHUNK,), jnp.float32),
            pltpu.VMEM((CHUNK,), jnp.float32),
            pltpu.VMEM((CHUNK,), jnp.float32),
            pltpu.VMEM((M * 16 * LANES,), jnp.float32),
            pltpu.VMEM((M * 4 * LANES,), jnp.int32),
            pltpu.VMEM((M * LANES,), jnp.float32),
            pltpu.VMEM((M * K * 3 * LANES,), jnp.float32),
            pltpu.VMEM((IDX_PER_CHUNK,), jnp.int32),
            pltpu.VMEM((IDX_PER_CHUNK,), jnp.float32),
            pltpu.VMEM((CHUNK * KP,), jnp.float32),
            pltpu.SemaphoreType.DMA,
        ],
    )
    cur = sc(px, py, pz, atab, btab, stab, dtab, sdf_ext)

    wpad = jnp.zeros((KP, O), jnp.float32).at[:K, :].set(weight.T)
    eye4 = jnp.eye(4, dtype=jnp.float32)
    wbig = jnp.einsum('pq,ko->pkqo', eye4, wpad).reshape(4 * KP, 4 * O)
    bbig = jnp.tile(bias, 4)[None, :]

    x = cur.reshape(tp // 4, 4 * KP)
    rows = tp // 4
    blk = 512
    out = pl.pallas_call(
        _tc_matmul_body,
        out_shape=jax.ShapeDtypeStruct((rows, 4 * O), jnp.float32),
        grid=(rows // blk,),
        in_specs=[
            pl.BlockSpec((blk, 4 * KP), lambda i: (i, 0)),
            pl.BlockSpec((4 * KP, 4 * O), lambda i: (0, 0)),
            pl.BlockSpec((1, 4 * O), lambda i: (0, 0)),
        ],
        out_specs=pl.BlockSpec((blk, 4 * O), lambda i: (i, 0)),
    )(x, wbig, bbig)

    out = out.reshape(B, npad, O)[:, :N, :]
    return out


# speedup vs baseline: 1.0001x; 1.0001x over previous
import functools

import jax
import jax.numpy as jnp
from jax import lax
from jax.experimental import pallas as pl
from jax.experimental.pallas import tpu as pltpu
from jax.experimental.pallas import tpu_sc as plsc

NDIM = 3
KS = 3
K = KS ** NDIM
KP = 32
DILATION = 0.05
MAX_DISTANCE = 1.0
SENT_VAL = 1e30

NTILES = 32
LANES = 16

CHUNK = 64
VPC = CHUNK // LANES
IDX_PER_CHUNK = CHUNK // LANES * 8 * K * LANES
GATHER_W = IDX_PER_CHUNK
NSTREAM = IDX_PER_CHUNK // GATHER_W


def _sc_kernel_body(nchunks, px_hbm, py_hbm, pz_hbm, atab_hbm, btab_hbm,
                    stab_hbm, dtab_hbm, sdf_hbm, out_hbm,
                    pxv, pyv, pzv, atab_v, btab_v, stab_v, dtab_v,
                    idx_buf, val_buf, cur_buf, gsem):
    M = 8
    wid = lax.axis_index("s") * 2 + lax.axis_index("c")
    b = wid // 8
    pbase = wid * (nchunks * CHUNK)

    pltpu.sync_copy(atab_hbm.at[pl.ds(b * (M * 16 * LANES), M * 16 * LANES)], atab_v)
    pltpu.sync_copy(btab_hbm.at[pl.ds(b * (M * 4 * LANES), M * 4 * LANES)], btab_v)
    pltpu.sync_copy(stab_hbm.at[pl.ds(b * (M * LANES), M * LANES)], stab_v)
    pltpu.sync_copy(dtab_hbm.at[pl.ds(b * (M * K * 3 * LANES), M * K * 3 * LANES)], dtab_v)

    zeros = jnp.zeros((LANES,), jnp.float32)

    def _zero(i, _):
        cur_buf[pl.ds(i * LANES, LANES)] = zeros
        return 0

    lax.fori_loop(0, CHUNK * KP // LANES, _zero, 0)

    iota = lax.iota(jnp.int32, LANES)
    iota_kp = iota * KP

    def chunk_body(g, _):
        off = pbase + g * CHUNK
        pltpu.sync_copy(px_hbm.at[pl.ds(off, CHUNK)], pxv)
        pltpu.sync_copy(py_hbm.at[pl.ds(off, CHUNK)], pyv)
        pltpu.sync_copy(pz_hbm.at[pl.ds(off, CHUNK)], pzv)

        def va_body(v, _):
            px = pxv[pl.ds(v * LANES, LANES)]
            py = pyv[pl.ds(v * LANES, LANES)]
            pz = pzv[pl.ds(v * LANES, LANES)]

            def m_body(m, _):
                ao = m * 16 * LANES
                a00 = atab_v[pl.ds(ao + 0 * LANES, LANES)]
                a01 = atab_v[pl.ds(ao + 1 * LANES, LANES)]
                a02 = atab_v[pl.ds(ao + 2 * LANES, LANES)]
                a10 = atab_v[pl.ds(ao + 3 * LANES, LANES)]
                a11 = atab_v[pl.ds(ao + 4 * LANES, LANES)]
                a12 = atab_v[pl.ds(ao + 5 * LANES, LANES)]
                a20 = atab_v[pl.ds(ao + 6 * LANES, LANES)]
                a21 = atab_v[pl.ds(ao + 7 * LANES, LANES)]
                a22 = atab_v[pl.ds(ao + 8 * LANES, LANES)]
                c0 = atab_v[pl.ds(ao + 9 * LANES, LANES)]
                c1 = atab_v[pl.ds(ao + 10 * LANES, LANES)]
                c2 = atab_v[pl.ds(ao + 11 * LANES, LANES)]
                dxf = atab_v[pl.ds(ao + 12 * LANES, LANES)]
                dyf = atab_v[pl.ds(ao + 13 * LANES, LANES)]
                dzf = atab_v[pl.ds(ao + 14 * LANES, LANES)]
                bo = m * 4 * LANES
                dyi = btab_v[pl.ds(bo + 0 * LANES, LANES)]
                dzi = btab_v[pl.ds(bo + 1 * LANES, LANES)]
                basev = btab_v[pl.ds(bo + 2 * LANES, LANES)]
                sentv = btab_v[pl.ds(bo + 3 * LANES, LANES)]

                xc = px * a00 + py * a01 + pz * a02 + c0
                yc = px * a10 + py * a11 + pz * a12 + c1
                zc = px * a20 + py * a21 + pz * a22 + c2

                do = m * (K * 3 * LANES)
                qbase = v * (8 * K * LANES) + m * (K * LANES)
                for k in range(K):
                    dx = dtab_v[pl.ds(do + (k * 3 + 0) * LANES, LANES)]
                    dy = dtab_v[pl.ds(do + (k * 3 + 1) * LANES, LANES)]
                    dz = dtab_v[pl.ds(do + (k * 3 + 2) * LANES, LANES)]
                    ux = xc + dx
                    uy = yc + dy
                    uz = zc + dz
                    inb = ((ux >= 0.0) & (ux < dxf)) & ((uy >= 0.0) & (uy < dyf))
                    inb = inb & ((uz >= 0.0) & (uz < dzf))
                    gx = ux.astype(jnp.int32)
                    gy = uy.astype(jnp.int32)
                    gz = uz.astype(jnp.int32)
                    flat = (gx * dyi + gy) * dzi + gz + basev
                    idx = jnp.where(inb, flat, sentv)
                    idx_buf[pl.ds(qbase + k * LANES, LANES)] = idx
                return 0

            lax.fori_loop(0, M, m_body, 0)
            return 0

        lax.fori_loop(0, VPC, va_body, 0)

        def fire(j, _):
            pltpu.async_copy(
                sdf_hbm.at[idx_buf.at[pl.ds(j * GATHER_W, GATHER_W)]],
                val_buf.at[pl.ds(j * GATHER_W, GATHER_W)], gsem)
            return 0

        lax.fori_loop(0, NSTREAM, fire, 0)

        def drain(j, _):
            pltpu.make_async_copy(
                sdf_hbm.at[idx_buf.at[pl.ds(j * GATHER_W, GATHER_W)]],
                val_buf.at[pl.ds(j * GATHER_W, GATHER_W)], gsem).wait()
            return 0

        lax.fori_loop(0, NSTREAM, drain, 0)

        def vb_body(v, _):
            svecs = [stab_v[pl.ds(m * LANES, LANES)] for m in range(M)]
            col = iota_kp + v * (LANES * KP)
            maxd = jnp.full((LANES,), MAX_DISTANCE, jnp.float32)
            for k in range(K):
                cur = maxd
                for m in range(M):
                    vo = v * (M * K * LANES) + m * (K * LANES) + k * LANES
                    cur = jnp.minimum(cur, val_buf[pl.ds(vo, LANES)] * svecs[m])
                plsc.store_scatter(cur_buf, [col + k], cur)
            return 0

        lax.fori_loop(0, VPC, vb_body, 0)

        pltpu.sync_copy(cur_buf, out_hbm.at[pl.ds((pbase + g * CHUNK) * KP,
                                                  CHUNK * KP)])
        return 0

    lax.fori_loop(0, nchunks, chunk_body, 0)


def _tc_matmul_body(x_ref, w_ref, b_ref, o_ref):
    o_ref[...] = jnp.dot(x_ref[...], w_ref[...],
                         preferred_element_type=jnp.float32) + b_ref[...]


def kernel(locs, idxs, poses, scales, sdf_data, sdf_offsets, sdf_shapes, weight, bias):
    B, N, _ = locs.shape
    M = idxs.shape[1]
    O = bias.shape[0]

    tiles_per_b = NTILES // B
    npad = ((N + tiles_per_b * CHUNK - 1) // (tiles_per_b * CHUNK)) * (tiles_per_b * CHUNK)
    nchunks = npad // (tiles_per_b * CHUNK)
    tp = B * npad

    p3 = jnp.pad(locs[..., :NDIM], ((0, 0), (0, npad - N), (0, 0)))
    px = p3[..., 0].reshape(-1)
    py = p3[..., 1].reshape(-1)
    pz = p3[..., 2].reshape(-1)

    t = poses[..., :NDIM]
    q = poses[..., NDIM:NDIM + 4]
    u = -q[..., :3]
    qw = q[..., 3]
    ux_, uy_, uz_ = u[..., 0], u[..., 1], u[..., 2]
    n2 = ux_ * ux_ + uy_ * uy_ + uz_ * uz_
    r00 = 1.0 - 2.0 * n2 + 2.0 * ux_ * ux_
    r11 = 1.0 - 2.0 * n2 + 2.0 * uy_ * uy_
    r22 = 1.0 - 2.0 * n2 + 2.0 * uz_ * uz_
    r01 = 2.0 * ux_ * uy_ - 2.0 * qw * uz_
    r02 = 2.0 * ux_ * uz_ + 2.0 * qw * uy_
    r10 = 2.0 * uy_ * ux_ + 2.0 * qw * uz_
    r12 = 2.0 * uy_ * uz_ - 2.0 * qw * ux_
    r20 = 2.0 * uz_ * ux_ - 2.0 * qw * uy_
    r21 = 2.0 * uz_ * uy_ + 2.0 * qw * ux_
    R = jnp.stack([r00, r01, r02, r10, r11, r12, r20, r21, r22],
                  axis=-1).reshape(B, M, 3, 3)

    sidx = idxs
    cell = sdf_shapes[sidx, NDIM]
    dims = sdf_shapes[sidx, :NDIM]
    base = sdf_offsets[sidx]
    inv = 1.0 / (scales * cell)

    A = R * inv[..., None, None]
    c = -jnp.sum(A * t[..., None, :], axis=-1)

    half = (KS - 1) // 2
    r = jnp.arange(-half, half + 1, dtype=jnp.float32) * DILATION
    ox, oy, oz = jnp.meshgrid(r, r, r, indexing='ij')
    offs = jnp.stack([ox.ravel(), oy.ravel(), oz.ravel()], axis=-1)
    d = jnp.sum(R[:, :, None, :, :] * offs[None, None, :, None, :], axis=-1)
    d = d * inv[..., None, None]

    atab = jnp.concatenate([A.reshape(B, M, 9), c, dims], axis=-1)
    atab = jnp.pad(atab, ((0, 0), (0, 0), (0, 1)))
    atab = jnp.broadcast_to(atab[..., None], (B, M, 16, LANES)).reshape(-1)

    sent = jnp.int32(sdf_data.shape[0])
    btab = jnp.stack([dims[..., 1].astype(jnp.int32),
                      dims[..., 2].astype(jnp.int32),
                      base.astype(jnp.int32),
                      jnp.broadcast_to(sent, (B, M))], axis=-1)
    btab = jnp.broadcast_to(btab[..., None], (B, M, 4, LANES)).reshape(-1)

    stab = jnp.broadcast_to(scales[..., None], (B, M, LANES)).reshape(-1)
    dtab = jnp.broadcast_to(d.reshape(B, M, K * 3)[..., None],
                            (B, M, K * 3, LANES)).reshape(-1)

    sdf_ext = jnp.concatenate(
        [sdf_data, jnp.full((16,), SENT_VAL, jnp.float32)])

    mesh = plsc.VectorSubcoreMesh(core_axis_name="c", subcore_axis_name="s")
    sc = pl.kernel(
        functools.partial(_sc_kernel_body, nchunks),
        out_type=jax.ShapeDtypeStruct((tp * KP,), jnp.float32),
        mesh=mesh,
        compiler_params=pltpu.CompilerParams(needs_layout_passes=False),
        scratch_types=[
            pltpu.VMEM((CHUNK,), jnp.float32),
            pltpu.VMEM((CHUNK,), jnp.float32),
            pltpu.VMEM((CHUNK,), jnp.float32),
            pltpu.VMEM((M * 16 * LANES,), jnp.float32),
            pltpu.VMEM((M * 4 * LANES,), jnp.int32),
            pltpu.VMEM((M * LANES,), jnp.float32),
            pltpu.VMEM((M * K * 3 * LANES,), jnp.float32),
            pltpu.VMEM((IDX_PER_CHUNK,), jnp.int32),
            pltpu.VMEM((IDX_PER_CHUNK,), jnp.float32),
            pltpu.VMEM((CHUNK * KP,), jnp.float32),
            pltpu.SemaphoreType.DMA,
        ],
    )
    cur = sc(px, py, pz, atab, btab, stab, dtab, sdf_ext)

    wpad = jnp.zeros((KP, O), jnp.float32).at[:K, :].set(weight.T)
    eye4 = jnp.eye(4, dtype=jnp.float32)
    wbig = jnp.einsum('pq,ko->pkqo', eye4, wpad).reshape(4 * KP, 4 * O)
    bbig = jnp.tile(bias, 4)[None, :]

    x = cur.reshape(tp // 4, 4 * KP)
    rows = tp // 4
    blk = 512
    out = pl.pallas_call(
        _tc_matmul_body,
        out_shape=jax.ShapeDtypeStruct((rows, 4 * O), jnp.float32),
        grid=(rows // blk,),
        in_specs=[
            pl.BlockSpec((blk, 4 * KP), lambda i: (i, 0)),
            pl.BlockSpec((4 * KP, 4 * O), lambda i: (0, 0)),
            pl.BlockSpec((1, 4 * O), lambda i: (0, 0)),
        ],
        out_specs=pl.BlockSpec((blk, 4 * O), lambda i: (i, 0)),
    )(x, wbig, bbig)

    out = out.reshape(B, npad, O)[:, :N, :]
    return out


# speedup vs baseline: 16.0714x; 16.0699x over previous
import functools

import jax
import jax.numpy as jnp
from jax import lax
from jax.experimental import pallas as pl
from jax.experimental.pallas import tpu as pltpu
from jax.experimental.pallas import tpu_sc as plsc

NDIM = 3
KS = 3
K = KS ** NDIM
KP = 32
DILATION = 0.05
MAX_DISTANCE = 1.0
SENT_VAL = 1e30

NTILES = 32
LANES = 16

CHUNK = 64
VPC = CHUNK // LANES
IDX_PER_CHUNK = CHUNK // LANES * 8 * K * LANES
GATHER_W = IDX_PER_CHUNK
NSTREAM = IDX_PER_CHUNK // GATHER_W


def _sc_kernel_body(nchunks, px_hbm, py_hbm, pz_hbm, atab_hbm, btab_hbm,
                    stab_hbm, dtab_hbm, sdf_hbm, out_hbm,
                    pxv, pyv, pzv, atab_v, btab_v, stab_v, dtab_v,
                    idx_buf, val_buf, cur_buf, sdf_sh, gsem):
    M = 8
    wid = lax.axis_index("s") * 2 + lax.axis_index("c")
    b = wid // 8
    pbase = wid * (nchunks * CHUNK)

    @pl.when(lax.axis_index("s") == 0)
    def _():
        pltpu.sync_copy(sdf_hbm, sdf_sh)

    plsc.subcore_barrier()

    pltpu.sync_copy(atab_hbm.at[pl.ds(b * (M * 16 * LANES), M * 16 * LANES)], atab_v)
    pltpu.sync_copy(btab_hbm.at[pl.ds(b * (M * 4 * LANES), M * 4 * LANES)], btab_v)
    pltpu.sync_copy(stab_hbm.at[pl.ds(b * (M * LANES), M * LANES)], stab_v)
    pltpu.sync_copy(dtab_hbm.at[pl.ds(b * (M * K * 3 * LANES), M * K * 3 * LANES)], dtab_v)

    zeros = jnp.zeros((LANES,), jnp.float32)

    def _zero(i, _):
        cur_buf[pl.ds(i * LANES, LANES)] = zeros
        return 0

    lax.fori_loop(0, CHUNK * KP // LANES, _zero, 0)

    iota = lax.iota(jnp.int32, LANES)
    iota_kp = iota * KP

    def chunk_body(g, _):
        off = pbase + g * CHUNK
        pltpu.sync_copy(px_hbm.at[pl.ds(off, CHUNK)], pxv)
        pltpu.sync_copy(py_hbm.at[pl.ds(off, CHUNK)], pyv)
        pltpu.sync_copy(pz_hbm.at[pl.ds(off, CHUNK)], pzv)

        def va_body(v, _):
            px = pxv[pl.ds(v * LANES, LANES)]
            py = pyv[pl.ds(v * LANES, LANES)]
            pz = pzv[pl.ds(v * LANES, LANES)]

            def m_body(m, _):
                ao = m * 16 * LANES
                a00 = atab_v[pl.ds(ao + 0 * LANES, LANES)]
                a01 = atab_v[pl.ds(ao + 1 * LANES, LANES)]
                a02 = atab_v[pl.ds(ao + 2 * LANES, LANES)]
                a10 = atab_v[pl.ds(ao + 3 * LANES, LANES)]
                a11 = atab_v[pl.ds(ao + 4 * LANES, LANES)]
                a12 = atab_v[pl.ds(ao + 5 * LANES, LANES)]
                a20 = atab_v[pl.ds(ao + 6 * LANES, LANES)]
                a21 = atab_v[pl.ds(ao + 7 * LANES, LANES)]
                a22 = atab_v[pl.ds(ao + 8 * LANES, LANES)]
                c0 = atab_v[pl.ds(ao + 9 * LANES, LANES)]
                c1 = atab_v[pl.ds(ao + 10 * LANES, LANES)]
                c2 = atab_v[pl.ds(ao + 11 * LANES, LANES)]
                dxf = atab_v[pl.ds(ao + 12 * LANES, LANES)]
                dyf = atab_v[pl.ds(ao + 13 * LANES, LANES)]
                dzf = atab_v[pl.ds(ao + 14 * LANES, LANES)]
                bo = m * 4 * LANES
                dyi = btab_v[pl.ds(bo + 0 * LANES, LANES)]
                dzi = btab_v[pl.ds(bo + 1 * LANES, LANES)]
                basev = btab_v[pl.ds(bo + 2 * LANES, LANES)]
                sentv = btab_v[pl.ds(bo + 3 * LANES, LANES)]

                xc = px * a00 + py * a01 + pz * a02 + c0
                yc = px * a10 + py * a11 + pz * a12 + c1
                zc = px * a20 + py * a21 + pz * a22 + c2

                do = m * (K * 3 * LANES)
                qbase = v * (8 * K * LANES) + m * (K * LANES)
                for k in range(K):
                    dx = dtab_v[pl.ds(do + (k * 3 + 0) * LANES, LANES)]
                    dy = dtab_v[pl.ds(do + (k * 3 + 1) * LANES, LANES)]
                    dz = dtab_v[pl.ds(do + (k * 3 + 2) * LANES, LANES)]
                    ux = xc + dx
                    uy = yc + dy
                    uz = zc + dz
                    inb = ((ux >= 0.0) & (ux < dxf)) & ((uy >= 0.0) & (uy < dyf))
                    inb = inb & ((uz >= 0.0) & (uz < dzf))
                    gx = ux.astype(jnp.int32)
                    gy = uy.astype(jnp.int32)
                    gz = uz.astype(jnp.int32)
                    flat = (gx * dyi + gy) * dzi + gz + basev
                    idx = jnp.where(inb, flat, sentv)
                    idx_buf[pl.ds(qbase + k * LANES, LANES)] = idx
                return 0

            lax.fori_loop(0, M, m_body, 0)
            return 0

        lax.fori_loop(0, VPC, va_body, 0)

        pltpu.async_copy(sdf_sh.at[idx_buf], val_buf, gsem)
        pltpu.make_async_copy(sdf_sh.at[idx_buf], val_buf, gsem).wait()

        def vb_body(v, _):
            svecs = [stab_v[pl.ds(m * LANES, LANES)] for m in range(M)]
            col = iota_kp + v * (LANES * KP)
            maxd = jnp.full((LANES,), MAX_DISTANCE, jnp.float32)
            for k in range(K):
                cur = maxd
                for m in range(M):
                    vo = v * (M * K * LANES) + m * (K * LANES) + k * LANES
                    cur = jnp.minimum(cur, val_buf[pl.ds(vo, LANES)] * svecs[m])
                plsc.store_scatter(cur_buf, [col + k], cur)
            return 0

        lax.fori_loop(0, VPC, vb_body, 0)

        pltpu.sync_copy(cur_buf, out_hbm.at[pl.ds((pbase + g * CHUNK) * KP,
                                                  CHUNK * KP)])
        return 0

    lax.fori_loop(0, nchunks, chunk_body, 0)


def _tc_matmul_body(x_ref, w_ref, b_ref, o_ref):
    o_ref[...] = jnp.dot(x_ref[...], w_ref[...],
                         preferred_element_type=jnp.float32) + b_ref[...]


def kernel(locs, idxs, poses, scales, sdf_data, sdf_offsets, sdf_shapes, weight, bias):
    B, N, _ = locs.shape
    M = idxs.shape[1]
    O = bias.shape[0]

    tiles_per_b = NTILES // B
    npad = ((N + tiles_per_b * CHUNK - 1) // (tiles_per_b * CHUNK)) * (tiles_per_b * CHUNK)
    nchunks = npad // (tiles_per_b * CHUNK)
    tp = B * npad

    p3 = jnp.pad(locs[..., :NDIM], ((0, 0), (0, npad - N), (0, 0)))
    px = p3[..., 0].reshape(-1)
    py = p3[..., 1].reshape(-1)
    pz = p3[..., 2].reshape(-1)

    t = poses[..., :NDIM]
    q = poses[..., NDIM:NDIM + 4]
    u = -q[..., :3]
    qw = q[..., 3]
    ux_, uy_, uz_ = u[..., 0], u[..., 1], u[..., 2]
    n2 = ux_ * ux_ + uy_ * uy_ + uz_ * uz_
    r00 = 1.0 - 2.0 * n2 + 2.0 * ux_ * ux_
    r11 = 1.0 - 2.0 * n2 + 2.0 * uy_ * uy_
    r22 = 1.0 - 2.0 * n2 + 2.0 * uz_ * uz_
    r01 = 2.0 * ux_ * uy_ - 2.0 * qw * uz_
    r02 = 2.0 * ux_ * uz_ + 2.0 * qw * uy_
    r10 = 2.0 * uy_ * ux_ + 2.0 * qw * uz_
    r12 = 2.0 * uy_ * uz_ - 2.0 * qw * ux_
    r20 = 2.0 * uz_ * ux_ - 2.0 * qw * uy_
    r21 = 2.0 * uz_ * uy_ + 2.0 * qw * ux_
    R = jnp.stack([r00, r01, r02, r10, r11, r12, r20, r21, r22],
                  axis=-1).reshape(B, M, 3, 3)

    sidx = idxs
    cell = sdf_shapes[sidx, NDIM]
    dims = sdf_shapes[sidx, :NDIM]
    base = sdf_offsets[sidx]
    inv = 1.0 / (scales * cell)

    A = R * inv[..., None, None]
    c = -jnp.sum(A * t[..., None, :], axis=-1)

    half = (KS - 1) // 2
    r = jnp.arange(-half, half + 1, dtype=jnp.float32) * DILATION
    ox, oy, oz = jnp.meshgrid(r, r, r, indexing='ij')
    offs = jnp.stack([ox.ravel(), oy.ravel(), oz.ravel()], axis=-1)
    d = jnp.sum(R[:, :, None, :, :] * offs[None, None, :, None, :], axis=-1)
    d = d * inv[..., None, None]

    atab = jnp.concatenate([A.reshape(B, M, 9), c, dims], axis=-1)
    atab = jnp.pad(atab, ((0, 0), (0, 0), (0, 1)))
    atab = jnp.broadcast_to(atab[..., None], (B, M, 16, LANES)).reshape(-1)

    sent = jnp.int32(sdf_data.shape[0])
    btab = jnp.stack([dims[..., 1].astype(jnp.int32),
                      dims[..., 2].astype(jnp.int32),
                      base.astype(jnp.int32),
                      jnp.broadcast_to(sent, (B, M))], axis=-1)
    btab = jnp.broadcast_to(btab[..., None], (B, M, 4, LANES)).reshape(-1)

    stab = jnp.broadcast_to(scales[..., None], (B, M, LANES)).reshape(-1)
    dtab = jnp.broadcast_to(d.reshape(B, M, K * 3)[..., None],
                            (B, M, K * 3, LANES)).reshape(-1)

    sdf_ext = jnp.concatenate(
        [sdf_data, jnp.full((16,), SENT_VAL, jnp.float32)])

    mesh = plsc.VectorSubcoreMesh(core_axis_name="c", subcore_axis_name="s")
    sc = pl.kernel(
        functools.partial(_sc_kernel_body, nchunks),
        out_type=jax.ShapeDtypeStruct((tp * KP,), jnp.float32),
        mesh=mesh,
        compiler_params=pltpu.CompilerParams(needs_layout_passes=False),
        scratch_types=[
            pltpu.VMEM((CHUNK,), jnp.float32),
            pltpu.VMEM((CHUNK,), jnp.float32),
            pltpu.VMEM((CHUNK,), jnp.float32),
            pltpu.VMEM((M * 16 * LANES,), jnp.float32),
            pltpu.VMEM((M * 4 * LANES,), jnp.int32),
            pltpu.VMEM((M * LANES,), jnp.float32),
            pltpu.VMEM((M * K * 3 * LANES,), jnp.float32),
            pltpu.VMEM((IDX_PER_CHUNK,), jnp.int32),
            pltpu.VMEM((IDX_PER_CHUNK,), jnp.float32),
            pltpu.VMEM((CHUNK * KP,), jnp.float32),
            pltpu.VMEM_SHARED((sdf_ext.shape[0],), jnp.float32),
            pltpu.SemaphoreType.DMA,
        ],
    )
    cur = sc(px, py, pz, atab, btab, stab, dtab, sdf_ext)

    wpad = jnp.zeros((KP, O), jnp.float32).at[:K, :].set(weight.T)
    eye4 = jnp.eye(4, dtype=jnp.float32)
    wbig = jnp.einsum('pq,ko->pkqo', eye4, wpad).reshape(4 * KP, 4 * O)
    bbig = jnp.tile(bias, 4)[None, :]

    x = cur.reshape(tp // 4, 4 * KP)
    rows = tp // 4
    blk = 512
    out = pl.pallas_call(
        _tc_matmul_body,
        out_shape=jax.ShapeDtypeStruct((rows, 4 * O), jnp.float32),
        grid=(rows // blk,),
        in_specs=[
            pl.BlockSpec((blk, 4 * KP), lambda i: (i, 0)),
            pl.BlockSpec((4 * KP, 4 * O), lambda i: (0, 0)),
            pl.BlockSpec((1, 4 * O), lambda i: (0, 0)),
        ],
        out_specs=pl.BlockSpec((blk, 4 * O), lambda i: (i, 0)),
    )(x, wbig, bbig)

    out = out.reshape(B, npad, O)[:, :N, :]
    return out


# speedup vs baseline: 16.1022x; 1.0019x over previous
import functools

import jax
import jax.numpy as jnp
from jax import lax
from jax.experimental import pallas as pl
from jax.experimental.pallas import tpu as pltpu
from jax.experimental.pallas import tpu_sc as plsc

NDIM = 3
KS = 3
K = KS ** NDIM
KP = 32
DILATION = 0.05
MAX_DISTANCE = 1.0
SENT_VAL = 1e30

NTILES = 32
LANES = 16

CHUNK = 32
VPC = CHUNK // LANES
IDX_PER_CHUNK = CHUNK // LANES * 8 * K * LANES
GATHER_W = IDX_PER_CHUNK
NSTREAM = IDX_PER_CHUNK // GATHER_W


def _sc_kernel_body(nchunks, px_hbm, py_hbm, pz_hbm, atab_hbm, btab_hbm,
                    stab_hbm, dtab_hbm, sdf_hbm, out_hbm,
                    pxv, pyv, pzv, atab_v, btab_v, stab_v, dtab_v,
                    idx_buf0, idx_buf1, val_buf0, val_buf1, cur_buf, sdf_sh,
                    gsem):
    M = 8
    wid = lax.axis_index("s") * 2 + lax.axis_index("c")
    b = wid // 8
    pbase = wid * (nchunks * CHUNK)

    @pl.when(lax.axis_index("s") == 0)
    def _():
        pltpu.sync_copy(sdf_hbm, sdf_sh)

    plsc.subcore_barrier()

    pltpu.sync_copy(atab_hbm.at[pl.ds(b * (M * 16 * LANES), M * 16 * LANES)], atab_v)
    pltpu.sync_copy(btab_hbm.at[pl.ds(b * (M * 4 * LANES), M * 4 * LANES)], btab_v)
    pltpu.sync_copy(stab_hbm.at[pl.ds(b * (M * LANES), M * LANES)], stab_v)
    pltpu.sync_copy(dtab_hbm.at[pl.ds(b * (M * K * 3 * LANES), M * K * 3 * LANES)], dtab_v)

    zeros = jnp.zeros((LANES,), jnp.float32)

    def _zero(i, _):
        cur_buf[pl.ds(i * LANES, LANES)] = zeros
        return 0

    lax.fori_loop(0, CHUNK * KP // LANES, _zero, 0)

    iota = lax.iota(jnp.int32, LANES)
    iota_kp = iota * KP

    def phase_a(g, idx_buf):
        off = pbase + g * CHUNK
        pltpu.sync_copy(px_hbm.at[pl.ds(off, CHUNK)], pxv)
        pltpu.sync_copy(py_hbm.at[pl.ds(off, CHUNK)], pyv)
        pltpu.sync_copy(pz_hbm.at[pl.ds(off, CHUNK)], pzv)

        def va_body(v, _):
            px = pxv[pl.ds(v * LANES, LANES)]
            py = pyv[pl.ds(v * LANES, LANES)]
            pz = pzv[pl.ds(v * LANES, LANES)]

            def m_body(m, _):
                ao = m * 16 * LANES
                a00 = atab_v[pl.ds(ao + 0 * LANES, LANES)]
                a01 = atab_v[pl.ds(ao + 1 * LANES, LANES)]
                a02 = atab_v[pl.ds(ao + 2 * LANES, LANES)]
                a10 = atab_v[pl.ds(ao + 3 * LANES, LANES)]
                a11 = atab_v[pl.ds(ao + 4 * LANES, LANES)]
                a12 = atab_v[pl.ds(ao + 5 * LANES, LANES)]
                a20 = atab_v[pl.ds(ao + 6 * LANES, LANES)]
                a21 = atab_v[pl.ds(ao + 7 * LANES, LANES)]
                a22 = atab_v[pl.ds(ao + 8 * LANES, LANES)]
                c0 = atab_v[pl.ds(ao + 9 * LANES, LANES)]
                c1 = atab_v[pl.ds(ao + 10 * LANES, LANES)]
                c2 = atab_v[pl.ds(ao + 11 * LANES, LANES)]
                dxf = atab_v[pl.ds(ao + 12 * LANES, LANES)]
                dyf = atab_v[pl.ds(ao + 13 * LANES, LANES)]
                dzf = atab_v[pl.ds(ao + 14 * LANES, LANES)]
                bo = m * 4 * LANES
                dyi = btab_v[pl.ds(bo + 0 * LANES, LANES)]
                dzi = btab_v[pl.ds(bo + 1 * LANES, LANES)]
                basev = btab_v[pl.ds(bo + 2 * LANES, LANES)]
                sentv = btab_v[pl.ds(bo + 3 * LANES, LANES)]

                xc = px * a00 + py * a01 + pz * a02 + c0
                yc = px * a10 + py * a11 + pz * a12 + c1
                zc = px * a20 + py * a21 + pz * a22 + c2

                do = m * (K * 3 * LANES)
                qbase = v * (8 * K * LANES) + m * (K * LANES)
                for k in range(K):
                    dx = dtab_v[pl.ds(do + (k * 3 + 0) * LANES, LANES)]
                    dy = dtab_v[pl.ds(do + (k * 3 + 1) * LANES, LANES)]
                    dz = dtab_v[pl.ds(do + (k * 3 + 2) * LANES, LANES)]
                    ux = xc + dx
                    uy = yc + dy
                    uz = zc + dz
                    inb = ((ux >= 0.0) & (ux < dxf)) & ((uy >= 0.0) & (uy < dyf))
                    inb = inb & ((uz >= 0.0) & (uz < dzf))
                    gx = ux.astype(jnp.int32)
                    gy = uy.astype(jnp.int32)
                    gz = uz.astype(jnp.int32)
                    flat = (gx * dyi + gy) * dzi + gz + basev
                    idx = jnp.where(inb, flat, sentv)
                    idx_buf[pl.ds(qbase + k * LANES, LANES)] = idx
                return 0

            lax.fori_loop(0, M, m_body, 0)
            return 0

        lax.fori_loop(0, VPC, va_body, 0)

    def phase_b(g, val_buf):
        def vb_body(v, _):
            svecs = [stab_v[pl.ds(m * LANES, LANES)] for m in range(M)]
            col = iota_kp + v * (LANES * KP)
            maxd = jnp.full((LANES,), MAX_DISTANCE, jnp.float32)
            for k in range(K):
                cur = maxd
                for m in range(M):
                    vo = v * (M * K * LANES) + m * (K * LANES) + k * LANES
                    cur = jnp.minimum(cur, val_buf[pl.ds(vo, LANES)] * svecs[m])
                plsc.store_scatter(cur_buf, [col + k], cur)
            return 0

        lax.fori_loop(0, VPC, vb_body, 0)

        pltpu.sync_copy(cur_buf, out_hbm.at[pl.ds((pbase + g * CHUNK) * KP,
                                                  CHUNK * KP)])

    def fire(idx_buf, val_buf):
        pltpu.async_copy(sdf_sh.at[idx_buf], val_buf, gsem)

    def wait(idx_buf, val_buf):
        pltpu.make_async_copy(sdf_sh.at[idx_buf], val_buf, gsem).wait()

    npairs = nchunks // 2
    phase_a(0, idx_buf0)
    fire(idx_buf0, val_buf0)

    def pair_body(h, _):
        g0 = 2 * h
        phase_a(g0 + 1, idx_buf1)
        fire(idx_buf1, val_buf1)
        wait(idx_buf0, val_buf0)
        phase_b(g0, val_buf0)

        @pl.when(h + 1 < npairs)
        def _():
            phase_a(g0 + 2, idx_buf0)
            fire(idx_buf0, val_buf0)

        wait(idx_buf1, val_buf1)
        phase_b(g0 + 1, val_buf1)
        return 0

    lax.fori_loop(0, npairs, pair_body, 0)


def _tc_matmul_body(x_ref, w_ref, b_ref, o_ref):
    o_ref[...] = jnp.dot(x_ref[...], w_ref[...],
                         preferred_element_type=jnp.float32) + b_ref[...]


def kernel(locs, idxs, poses, scales, sdf_data, sdf_offsets, sdf_shapes, weight, bias):
    B, N, _ = locs.shape
    M = idxs.shape[1]
    O = bias.shape[0]

    tiles_per_b = NTILES // B
    npad = ((N + tiles_per_b * CHUNK - 1) // (tiles_per_b * CHUNK)) * (tiles_per_b * CHUNK)
    nchunks = npad // (tiles_per_b * CHUNK)
    if nchunks % 2:
        nchunks += 1
        npad = nchunks * tiles_per_b * CHUNK
    tp = B * npad

    p3 = jnp.pad(locs[..., :NDIM], ((0, 0), (0, npad - N), (0, 0)))
    px = p3[..., 0].reshape(-1)
    py = p3[..., 1].reshape(-1)
    pz = p3[..., 2].reshape(-1)

    t = poses[..., :NDIM]
    q = poses[..., NDIM:NDIM + 4]
    u = -q[..., :3]
    qw = q[..., 3]
    ux_, uy_, uz_ = u[..., 0], u[..., 1], u[..., 2]
    n2 = ux_ * ux_ + uy_ * uy_ + uz_ * uz_
    r00 = 1.0 - 2.0 * n2 + 2.0 * ux_ * ux_
    r11 = 1.0 - 2.0 * n2 + 2.0 * uy_ * uy_
    r22 = 1.0 - 2.0 * n2 + 2.0 * uz_ * uz_
    r01 = 2.0 * ux_ * uy_ - 2.0 * qw * uz_
    r02 = 2.0 * ux_ * uz_ + 2.0 * qw * uy_
    r10 = 2.0 * uy_ * ux_ + 2.0 * qw * uz_
    r12 = 2.0 * uy_ * uz_ - 2.0 * qw * ux_
    r20 = 2.0 * uz_ * ux_ - 2.0 * qw * uy_
    r21 = 2.0 * uz_ * uy_ + 2.0 * qw * ux_
    R = jnp.stack([r00, r01, r02, r10, r11, r12, r20, r21, r22],
                  axis=-1).reshape(B, M, 3, 3)

    sidx = idxs
    cell = sdf_shapes[sidx, NDIM]
    dims = sdf_shapes[sidx, :NDIM]
    base = sdf_offsets[sidx]
    inv = 1.0 / (scales * cell)

    A = R * inv[..., None, None]
    c = -jnp.sum(A * t[..., None, :], axis=-1)

    half = (KS - 1) // 2
    r = jnp.arange(-half, half + 1, dtype=jnp.float32) * DILATION
    ox, oy, oz = jnp.meshgrid(r, r, r, indexing='ij')
    offs = jnp.stack([ox.ravel(), oy.ravel(), oz.ravel()], axis=-1)
    d = jnp.sum(R[:, :, None, :, :] * offs[None, None, :, None, :], axis=-1)
    d = d * inv[..., None, None]

    atab = jnp.concatenate([A.reshape(B, M, 9), c, dims], axis=-1)
    atab = jnp.pad(atab, ((0, 0), (0, 0), (0, 1)))
    atab = jnp.broadcast_to(atab[..., None], (B, M, 16, LANES)).reshape(-1)

    sent = jnp.int32(sdf_data.shape[0])
    btab = jnp.stack([dims[..., 1].astype(jnp.int32),
                      dims[..., 2].astype(jnp.int32),
                      base.astype(jnp.int32),
                      jnp.broadcast_to(sent, (B, M))], axis=-1)
    btab = jnp.broadcast_to(btab[..., None], (B, M, 4, LANES)).reshape(-1)

    stab = jnp.broadcast_to(scales[..., None], (B, M, LANES)).reshape(-1)
    dtab = jnp.broadcast_to(d.reshape(B, M, K * 3)[..., None],
                            (B, M, K * 3, LANES)).reshape(-1)

    sdf_ext = jnp.concatenate(
        [sdf_data, jnp.full((16,), SENT_VAL, jnp.float32)])

    mesh = plsc.VectorSubcoreMesh(core_axis_name="c", subcore_axis_name="s")
    sc = pl.kernel(
        functools.partial(_sc_kernel_body, nchunks),
        out_type=jax.ShapeDtypeStruct((tp * KP,), jnp.float32),
        mesh=mesh,
        compiler_params=pltpu.CompilerParams(needs_layout_passes=False),
        scratch_types=[
            pltpu.VMEM((CHUNK,), jnp.float32),
            pltpu.VMEM((CHUNK,), jnp.float32),
            pltpu.VMEM((CHUNK,), jnp.float32),
            pltpu.VMEM((M * 16 * LANES,), jnp.float32),
            pltpu.VMEM((M * 4 * LANES,), jnp.int32),
            pltpu.VMEM((M * LANES,), jnp.float32),
            pltpu.VMEM((M * K * 3 * LANES,), jnp.float32),
            pltpu.VMEM((IDX_PER_CHUNK,), jnp.int32),
            pltpu.VMEM((IDX_PER_CHUNK,), jnp.int32),
            pltpu.VMEM((IDX_PER_CHUNK,), jnp.float32),
            pltpu.VMEM((IDX_PER_CHUNK,), jnp.float32),
            pltpu.VMEM((CHUNK * KP,), jnp.float32),
            pltpu.VMEM_SHARED((sdf_ext.shape[0],), jnp.float32),
            pltpu.SemaphoreType.DMA,
        ],
    )
    cur = sc(px, py, pz, atab, btab, stab, dtab, sdf_ext)

    wpad = jnp.zeros((KP, O), jnp.float32).at[:K, :].set(weight.T)
    eye4 = jnp.eye(4, dtype=jnp.float32)
    wbig = jnp.einsum('pq,ko->pkqo', eye4, wpad).reshape(4 * KP, 4 * O)
    bbig = jnp.tile(bias, 4)[None, :]

    x = cur.reshape(tp // 4, 4 * KP)
    rows = tp // 4
    blk = 512
    out = pl.pallas_call(
        _tc_matmul_body,
        out_shape=jax.ShapeDtypeStruct((rows, 4 * O), jnp.float32),
        grid=(rows // blk,),
        in_specs=[
            pl.BlockSpec((blk, 4 * KP), lambda i: (i, 0)),
            pl.BlockSpec((4 * KP, 4 * O), lambda i: (0, 0)),
            pl.BlockSpec((1, 4 * O), lambda i: (0, 0)),
        ],
        out_specs=pl.BlockSpec((blk, 4 * O), lambda i: (i, 0)),
    )(x, wbig, bbig)

    out = out.reshape(B, npad, O)[:, :N, :]
    return out


# speedup vs baseline: 16.1040x; 1.0001x over previous
import functools

import jax
import jax.numpy as jnp
from jax import lax
from jax.experimental import pallas as pl
from jax.experimental.pallas import tpu as pltpu
from jax.experimental.pallas import tpu_sc as plsc

NDIM = 3
KS = 3
K = KS ** NDIM
KP = 32
DILATION = 0.05
MAX_DISTANCE = 1.0
SENT_VAL = 1e30

NTILES = 32
LANES = 16

CHUNK = 32
VPC = CHUNK // LANES
IDX_PER_CHUNK = CHUNK // LANES * 8 * K * LANES


def _sc_kernel_body(nchunks, px_hbm, py_hbm, pz_hbm, atab_hbm, btab_hbm,
                    stab_hbm, dtab_hbm, sdf_hbm, out_hbm,
                    pxv, pyv, pzv, atab_v, btab_v, stab_v, dtab_v,
                    idx_buf0, idx_buf1, val_buf0, val_buf1, cur_buf, sdf_sh,
                    gsem):
    M = 8
    wid = lax.axis_index("s") * 2 + lax.axis_index("c")
    b = wid // 8
    pbase = wid * (nchunks * CHUNK)

    @pl.when(lax.axis_index("s") == 0)
    def _():
        pltpu.sync_copy(sdf_hbm, sdf_sh)

    plsc.subcore_barrier()

    pltpu.sync_copy(atab_hbm.at[pl.ds(b * (M * 16 * LANES), M * 16 * LANES)], atab_v)
    pltpu.sync_copy(btab_hbm.at[pl.ds(b * (M * 4 * LANES), M * 4 * LANES)], btab_v)
    pltpu.sync_copy(stab_hbm.at[pl.ds(b * (M * LANES), M * LANES)], stab_v)
    pltpu.sync_copy(dtab_hbm.at[pl.ds(b * (M * K * 3 * LANES), M * K * 3 * LANES)], dtab_v)

    zeros = jnp.zeros((LANES,), jnp.float32)

    def _zero(i, _):
        cur_buf[pl.ds(i * LANES, LANES)] = zeros
        return 0

    lax.fori_loop(0, CHUNK * KP // LANES, _zero, 0)

    iota = lax.iota(jnp.int32, LANES)
    iota_kp = iota * KP

    def phase_a(g, idx_buf):
        off = pbase + g * CHUNK
        pltpu.sync_copy(px_hbm.at[pl.ds(off, CHUNK)], pxv)
        pltpu.sync_copy(py_hbm.at[pl.ds(off, CHUNK)], pyv)
        pltpu.sync_copy(pz_hbm.at[pl.ds(off, CHUNK)], pzv)

        def va_body(v, _):
            px = pxv[pl.ds(v * LANES, LANES)]
            py = pyv[pl.ds(v * LANES, LANES)]
            pz = pzv[pl.ds(v * LANES, LANES)]

            def m_body(m, _):
                ao = m * 16 * LANES
                a00 = atab_v[pl.ds(ao + 0 * LANES, LANES)]
                a01 = atab_v[pl.ds(ao + 1 * LANES, LANES)]
                a02 = atab_v[pl.ds(ao + 2 * LANES, LANES)]
                a10 = atab_v[pl.ds(ao + 3 * LANES, LANES)]
                a11 = atab_v[pl.ds(ao + 4 * LANES, LANES)]
                a12 = atab_v[pl.ds(ao + 5 * LANES, LANES)]
                a20 = atab_v[pl.ds(ao + 6 * LANES, LANES)]
                a21 = atab_v[pl.ds(ao + 7 * LANES, LANES)]
                a22 = atab_v[pl.ds(ao + 8 * LANES, LANES)]
                c0 = atab_v[pl.ds(ao + 9 * LANES, LANES)]
                c1 = atab_v[pl.ds(ao + 10 * LANES, LANES)]
                c2 = atab_v[pl.ds(ao + 11 * LANES, LANES)]
                dxf = atab_v[pl.ds(ao + 12 * LANES, LANES)]
                dyf = atab_v[pl.ds(ao + 13 * LANES, LANES)]
                dzf = atab_v[pl.ds(ao + 14 * LANES, LANES)]
                bo = m * 4 * LANES
                dyi = btab_v[pl.ds(bo + 0 * LANES, LANES)]
                dzi = btab_v[pl.ds(bo + 1 * LANES, LANES)]
                basev = btab_v[pl.ds(bo + 2 * LANES, LANES)]
                sentv = btab_v[pl.ds(bo + 3 * LANES, LANES)]

                xc = px * a00 + py * a01 + pz * a02 + c0
                yc = px * a10 + py * a11 + pz * a12 + c1
                zc = px * a20 + py * a21 + pz * a22 + c2

                do = m * (K * 3 * LANES)
                qbase = v * (8 * K * LANES) + m * (K * LANES)
                for k in range(K):
                    dx = dtab_v[pl.ds(do + (k * 3 + 0) * LANES, LANES)]
                    dy = dtab_v[pl.ds(do + (k * 3 + 1) * LANES, LANES)]
                    dz = dtab_v[pl.ds(do + (k * 3 + 2) * LANES, LANES)]
                    ux = xc + dx
                    uy = yc + dy
                    uz = zc + dz
                    inb = ((ux >= 0.0) & (ux < dxf)) & ((uy >= 0.0) & (uy < dyf))
                    inb = inb & ((uz >= 0.0) & (uz < dzf))
                    gx = ux.astype(jnp.int32)
                    gy = uy.astype(jnp.int32)
                    gz = uz.astype(jnp.int32)
                    flat = (gx * dyi + gy) * dzi + gz + basev
                    idx = jnp.where(inb, flat, sentv)
                    idx_buf[pl.ds(qbase + k * LANES, LANES)] = idx
                return 0

            lax.fori_loop(0, M, m_body, 0)
            return 0

        lax.fori_loop(0, VPC, va_body, 0)

    def phase_b(g, val_buf):
        def vb_body(v, _):
            svecs = [stab_v[pl.ds(m * LANES, LANES)] for m in range(M)]
            col = iota_kp + v * (LANES * KP)
            maxd = jnp.full((LANES,), MAX_DISTANCE, jnp.float32)
            for k in range(K):
                cur = maxd
                for m in range(M):
                    vo = v * (M * K * LANES) + m * (K * LANES) + k * LANES
                    cur = jnp.minimum(cur, val_buf[pl.ds(vo, LANES)] * svecs[m])
                plsc.store_scatter(cur_buf, [col + k], cur)
            return 0

        lax.fori_loop(0, VPC, vb_body, 0)

        pltpu.sync_copy(cur_buf, out_hbm.at[pl.ds((pbase + g * CHUNK) * KP,
                                                  CHUNK * KP)])

    def fire(idx_buf, val_buf):
        pltpu.async_copy(sdf_sh.at[idx_buf], val_buf, gsem)

    def wait(idx_buf, val_buf):
        pltpu.make_async_copy(sdf_sh.at[idx_buf], val_buf, gsem).wait()

    npairs = nchunks // 2
    phase_a(0, idx_buf0)
    fire(idx_buf0, val_buf0)

    def pair_body(h, _):
        g0 = 2 * h
        phase_a(g0 + 1, idx_buf1)
        fire(idx_buf1, val_buf1)
        wait(idx_buf0, val_buf0)
        phase_b(g0, val_buf0)

        @pl.when(h + 1 < npairs)
        def _():
            phase_a(g0 + 2, idx_buf0)
            fire(idx_buf0, val_buf0)

        wait(idx_buf1, val_buf1)
        phase_b(g0 + 1, val_buf1)
        return 0

    lax.fori_loop(0, npairs, pair_body, 0)


def _tc_matmul_body(x_ref, w_ref, b_ref, o_ref):
    o_ref[...] = jnp.dot(x_ref[...], w_ref[...],
                         preferred_element_type=jnp.float32) + b_ref[...]


def kernel(locs, idxs, poses, scales, sdf_data, sdf_offsets, sdf_shapes, weight, bias):
    B, N, _ = locs.shape
    M = idxs.shape[1]
    O = bias.shape[0]

    tiles_per_b = NTILES // B
    npad = ((N + tiles_per_b * CHUNK - 1) // (tiles_per_b * CHUNK)) * (tiles_per_b * CHUNK)
    nchunks = npad // (tiles_per_b * CHUNK)
    if nchunks % 2:
        nchunks += 1
        npad = nchunks * tiles_per_b * CHUNK
    tp = B * npad

    p3 = jnp.pad(locs[..., :NDIM], ((0, 0), (0, npad - N), (0, 0)))
    px = p3[..., 0].reshape(-1)
    py = p3[..., 1].reshape(-1)
    pz = p3[..., 2].reshape(-1)

    t = poses[..., :NDIM]
    q = poses[..., NDIM:NDIM + 4]
    u = -q[..., :3]
    qw = q[..., 3]
    ux_, uy_, uz_ = u[..., 0], u[..., 1], u[..., 2]
    n2 = ux_ * ux_ + uy_ * uy_ + uz_ * uz_
    r00 = 1.0 - 2.0 * n2 + 2.0 * ux_ * ux_
    r11 = 1.0 - 2.0 * n2 + 2.0 * uy_ * uy_
    r22 = 1.0 - 2.0 * n2 + 2.0 * uz_ * uz_
    r01 = 2.0 * ux_ * uy_ - 2.0 * qw * uz_
    r02 = 2.0 * ux_ * uz_ + 2.0 * qw * uy_
    r10 = 2.0 * uy_ * ux_ + 2.0 * qw * uz_
    r12 = 2.0 * uy_ * uz_ - 2.0 * qw * ux_
    r20 = 2.0 * uz_ * ux_ - 2.0 * qw * uy_
    r21 = 2.0 * uz_ * uy_ + 2.0 * qw * ux_
    R = jnp.stack([r00, r01, r02, r10, r11, r12, r20, r21, r22],
                  axis=-1).reshape(B, M, 3, 3)

    sidx = idxs
    cell = sdf_shapes[sidx, NDIM]
    dims = sdf_shapes[sidx, :NDIM]
    base = sdf_offsets[sidx]
    inv = 1.0 / (scales * cell)

    A = R * inv[..., None, None]
    c = -jnp.sum(A * t[..., None, :], axis=-1)

    half = (KS - 1) // 2
    r = jnp.arange(-half, half + 1, dtype=jnp.float32) * DILATION
    ox, oy, oz = jnp.meshgrid(r, r, r, indexing='ij')
    offs = jnp.stack([ox.ravel(), oy.ravel(), oz.ravel()], axis=-1)
    d = jnp.sum(R[:, :, None, :, :] * offs[None, None, :, None, :], axis=-1)
    d = d * inv[..., None, None]

    atab = jnp.concatenate([A.reshape(B, M, 9), c, dims], axis=-1)
    atab = jnp.pad(atab, ((0, 0), (0, 0), (0, 1)))
    atab = jnp.broadcast_to(atab[..., None], (B, M, 16, LANES)).reshape(-1)

    sent = jnp.int32(sdf_data.shape[0])
    btab = jnp.stack([dims[..., 1].astype(jnp.int32),
                      dims[..., 2].astype(jnp.int32),
                      base.astype(jnp.int32),
                      jnp.broadcast_to(sent, (B, M))], axis=-1)
    btab = jnp.broadcast_to(btab[..., None], (B, M, 4, LANES)).reshape(-1)

    stab = jnp.broadcast_to(scales[..., None], (B, M, LANES)).reshape(-1)
    dtab = jnp.broadcast_to(d.reshape(B, M, K * 3)[..., None],
                            (B, M, K * 3, LANES)).reshape(-1)

    sdf_ext = jnp.concatenate(
        [sdf_data, jnp.full((16,), SENT_VAL, jnp.float32)])

    mesh = plsc.VectorSubcoreMesh(core_axis_name="c", subcore_axis_name="s")
    sc = pl.kernel(
        functools.partial(_sc_kernel_body, nchunks),
        out_type=jax.ShapeDtypeStruct((tp * KP,), jnp.float32),
        mesh=mesh,
        compiler_params=pltpu.CompilerParams(needs_layout_passes=False),
        scratch_types=[
            pltpu.VMEM((CHUNK,), jnp.float32),
            pltpu.VMEM((CHUNK,), jnp.float32),
            pltpu.VMEM((CHUNK,), jnp.float32),
            pltpu.VMEM((M * 16 * LANES,), jnp.float32),
            pltpu.VMEM((M * 4 * LANES,), jnp.int32),
            pltpu.VMEM((M * LANES,), jnp.float32),
            pltpu.VMEM((M * K * 3 * LANES,), jnp.float32),
            pltpu.VMEM((IDX_PER_CHUNK,), jnp.int32),
            pltpu.VMEM((IDX_PER_CHUNK,), jnp.int32),
            pltpu.VMEM((IDX_PER_CHUNK,), jnp.float32),
            pltpu.VMEM((IDX_PER_CHUNK,), jnp.float32),
            pltpu.VMEM((CHUNK * KP,), jnp.float32),
            pltpu.VMEM_SHARED((sdf_ext.shape[0],), jnp.float32),
            pltpu.SemaphoreType.DMA,
        ],
    )
    cur = sc(px, py, pz, atab, btab, stab, dtab, sdf_ext)

    wpad = jnp.zeros((KP, O), jnp.float32).at[:K, :].set(weight.T)
    eye4 = jnp.eye(4, dtype=jnp.float32)
    wbig = jnp.einsum('pq,ko->pkqo', eye4, wpad).reshape(4 * KP, 4 * O)
    bbig = jnp.tile(bias, 4)[None, :]

    x = cur.reshape(tp // 4, 4 * KP)
    rows = tp // 4
    blk = 512
    out = pl.pallas_call(
        _tc_matmul_body,
        out_shape=jax.ShapeDtypeStruct((rows, 4 * O), jnp.float32),
        grid=(rows // blk,),
        in_specs=[
            pl.BlockSpec((blk, 4 * KP), lambda i: (i, 0)),
            pl.BlockSpec((4 * KP, 4 * O), lambda i: (0, 0)),
            pl.BlockSpec((1, 4 * O), lambda i: (0, 0)),
        ],
        out_specs=pl.BlockSpec((blk, 4 * O), lambda i: (i, 0)),
    )(x, wbig, bbig)

    out = out.reshape(B, npad, O)[:, :N, :]
    return out
